# baseline (device time: 660046 ns/iter reference)
import jax
import jax.numpy as jnp
from jax import lax
from jax.experimental import pallas as pl
from jax.experimental.pallas import tpu as pltpu

N_BLOCKS = 32
BLK_N = 8192 // N_BLOCKS
S_HALF = 1024
K_SHARD = 4096


def kernel(O, Wo):
    O2 = O.reshape(2048, K_SHARD)
    y = lax.axis_index("y")
    O_mine = lax.dynamic_slice(O2, (S_HALF * y, 0), (S_HALF, K_SHARD))
    O_theirs = lax.dynamic_slice(O2, (S_HALF * (1 - y), 0), (S_HALF, K_SHARD))

    def body(o_mine_ref, o_theirs_ref, wo_ref, out_ref,
             send_buf, recv_buf, send_sems, recv_sems):
        k = pl.program_id(0)
        my_x = lax.axis_index("x")
        my_y = lax.axis_index("y")
        my_z = lax.axis_index("z")
        partner = (my_x, 1 - my_y, my_z)

        @pl.when(k == 0)
        def _():
            barrier_sem = pltpu.get_barrier_semaphore()
            pl.semaphore_signal(
                barrier_sem, inc=1,
                device_id=partner, device_id_type=pl.DeviceIdType.MESH,
            )
            pl.semaphore_wait(barrier_sem, 1)

        slot = lax.rem(k, 2)

        out_ref[...] = jnp.dot(
            o_mine_ref[...], wo_ref[...], preferred_element_type=jnp.float32
        )
        send_buf[slot] = jnp.dot(
            o_theirs_ref[...], wo_ref[...], preferred_element_type=jnp.float32
        )

        rdma = pltpu.make_async_remote_copy(
            src_ref=send_buf.at[slot],
            dst_ref=recv_buf.at[slot],
            send_sem=send_sems.at[slot],
            recv_sem=recv_sems.at[slot],
            device_id=partner,
            device_id_type=pl.DeviceIdType.MESH,
        )
        rdma.start()
        rdma.wait()

        out_ref[...] += recv_buf[slot]

    out = pl.pallas_call(
        body,
        grid=(N_BLOCKS,),
        in_specs=[
            pl.BlockSpec((S_HALF, K_SHARD), lambda k: (0, 0)),
            pl.BlockSpec((S_HALF, K_SHARD), lambda k: (0, 0)),
            pl.BlockSpec((K_SHARD, BLK_N), lambda k: (0, k)),
        ],
        out_specs=pl.BlockSpec((S_HALF, BLK_N), lambda k: (0, k)),
        out_shape=jax.ShapeDtypeStruct((S_HALF, 8192), jnp.float32),
        scratch_shapes=[
            pltpu.VMEM((2, S_HALF, BLK_N), jnp.float32),
            pltpu.VMEM((2, S_HALF, BLK_N), jnp.float32),
            pltpu.SemaphoreType.DMA((2,)),
            pltpu.SemaphoreType.DMA((2,)),
        ],
        compiler_params=pltpu.CompilerParams(
            collective_id=0, dimension_semantics=("arbitrary",)
        ),
    )(O_mine, O_theirs, Wo)
    return out.reshape(1, S_HALF, 8192)


# device time: 455259 ns/iter; 1.4498x vs baseline; 1.4498x over previous
import jax
import jax.numpy as jnp
from jax import lax
from jax.experimental import pallas as pl
from jax.experimental.pallas import tpu as pltpu

N_BLOCKS = 32
BLK_N = 8192 // N_BLOCKS
S_HALF = 1024
K_SHARD = 4096


def kernel(O, Wo):
    O2 = O.reshape(2048, K_SHARD)
    y = lax.axis_index("y")
    O_mine = lax.dynamic_slice(O2, (S_HALF * y, 0), (S_HALF, K_SHARD))
    O_theirs = lax.dynamic_slice(O2, (S_HALF * (1 - y), 0), (S_HALF, K_SHARD))

    def body(o_mine_ref, o_theirs_ref, wo_ref, out_ref,
             keep_buf, send_buf, recv_buf, send_sems, recv_sems, credit_sem):
        k = pl.program_id(0)
        my_x = lax.axis_index("x")
        my_y = lax.axis_index("y")
        my_z = lax.axis_index("z")
        partner = (my_x, 1 - my_y, my_z)

        slot = lax.rem(k, 2)
        pslot = lax.rem(k + 1, 2)

        def mk(s):
            return pltpu.make_async_remote_copy(
                src_ref=send_buf.at[s],
                dst_ref=recv_buf.at[s],
                send_sem=send_sems.at[s],
                recv_sem=recv_sems.at[s],
                device_id=partner,
                device_id_type=pl.DeviceIdType.MESH,
            )

        @pl.when(k == 0)
        def _():
            barrier_sem = pltpu.get_barrier_semaphore()
            pl.semaphore_signal(
                barrier_sem, inc=1,
                device_id=partner, device_id_type=pl.DeviceIdType.MESH,
            )
            pl.semaphore_wait(barrier_sem, 1)

        @pl.when(k < N_BLOCKS)
        def _():
            @pl.when(k >= 2)
            def _():
                mk(slot).wait_send()

            keep_buf[slot] = jnp.dot(
                o_mine_ref[...], wo_ref[...],
                preferred_element_type=jnp.float32,
            )
            send_buf[slot] = jnp.dot(
                o_theirs_ref[...], wo_ref[...],
                preferred_element_type=jnp.float32,
            )

            @pl.when(k >= 2)
            def _():
                pl.semaphore_wait(credit_sem, 1)

            mk(slot).start()

        @pl.when(k >= 1)
        def _():
            mk(pslot).wait_recv()
            out_ref[...] = keep_buf[pslot] + recv_buf[pslot]
            pl.semaphore_signal(
                credit_sem, inc=1,
                device_id=partner, device_id_type=pl.DeviceIdType.MESH,
            )

        @pl.when(k == N_BLOCKS)
        def _():
            pl.semaphore_wait(credit_sem, 2)
            mk(0).wait_send()
            mk(1).wait_send()

    last = N_BLOCKS - 1
    out = pl.pallas_call(
        body,
        grid=(N_BLOCKS + 1,),
        in_specs=[
            pl.BlockSpec((S_HALF, K_SHARD), lambda k: (0, 0)),
            pl.BlockSpec((S_HALF, K_SHARD), lambda k: (0, 0)),
            pl.BlockSpec((K_SHARD, BLK_N), lambda k: (0, jnp.minimum(k, last))),
        ],
        out_specs=pl.BlockSpec(
            (S_HALF, BLK_N), lambda k: (0, jnp.maximum(k - 1, 0))
        ),
        out_shape=jax.ShapeDtypeStruct((S_HALF, 8192), jnp.float32),
        scratch_shapes=[
            pltpu.VMEM((2, S_HALF, BLK_N), jnp.float32),
            pltpu.VMEM((2, S_HALF, BLK_N), jnp.float32),
            pltpu.VMEM((2, S_HALF, BLK_N), jnp.float32),
            pltpu.SemaphoreType.DMA((2,)),
            pltpu.SemaphoreType.DMA((2,)),
            pltpu.SemaphoreType.REGULAR,
        ],
        compiler_params=pltpu.CompilerParams(
            collective_id=0, dimension_semantics=("arbitrary",)
        ),
    )(O_mine, O_theirs, Wo)
    return out.reshape(1, S_HALF, 8192)


# device time: 454099 ns/iter; 1.4535x vs baseline; 1.0026x over previous
import jax
import jax.numpy as jnp
from jax import lax
from jax.experimental import pallas as pl
from jax.experimental.pallas import tpu as pltpu

N_BLOCKS = 32
BLK_N = 8192 // N_BLOCKS
S_HALF = 1024
K_SHARD = 4096


def kernel(O, Wo):
    O2 = O.reshape(2048, K_SHARD)
    NSLOT = 4

    def body(o_ref, wo_ref, out_ref,
             keep_buf, send_buf, recv_buf, send_sems, recv_sems, credit_sem):
        k = pl.program_id(0)
        my_x = lax.axis_index("x")
        my_y = lax.axis_index("y")
        my_z = lax.axis_index("z")
        partner = (my_x, 1 - my_y, my_z)

        slot = lax.rem(k, NSLOT)
        pslot = lax.rem(k + NSLOT - 1, NSLOT)

        def mk(s):
            return pltpu.make_async_remote_copy(
                src_ref=send_buf.at[s],
                dst_ref=recv_buf.at[s],
                send_sem=send_sems.at[s],
                recv_sem=recv_sems.at[s],
                device_id=partner,
                device_id_type=pl.DeviceIdType.MESH,
            )

        @pl.when(k == 0)
        def _():
            barrier_sem = pltpu.get_barrier_semaphore()
            pl.semaphore_signal(
                barrier_sem, inc=1,
                device_id=partner, device_id_type=pl.DeviceIdType.MESH,
            )
            pl.semaphore_wait(barrier_sem, 1)

        @pl.when(k < N_BLOCKS)
        def _():
            @pl.when(k >= NSLOT)
            def _():
                mk(slot).wait_send()

            keep_buf[slot] = jnp.dot(
                o_ref[pl.ds(S_HALF * my_y, S_HALF), :], wo_ref[...],
                preferred_element_type=jnp.float32,
            )
            send_buf[slot] = jnp.dot(
                o_ref[pl.ds(S_HALF * (1 - my_y), S_HALF), :], wo_ref[...],
                preferred_element_type=jnp.float32,
            )

            @pl.when(k >= NSLOT)
            def _():
                pl.semaphore_wait(credit_sem, 1)

            mk(slot).start()

        @pl.when(k >= 1)
        def _():
            mk(pslot).wait_recv()
            out_ref[...] = keep_buf[pslot] + recv_buf[pslot]
            pl.semaphore_signal(
                credit_sem, inc=1,
                device_id=partner, device_id_type=pl.DeviceIdType.MESH,
            )

        @pl.when(k == N_BLOCKS)
        def _():
            pl.semaphore_wait(credit_sem, NSLOT)
            for s in range(NSLOT):
                mk(s).wait_send()

    last = N_BLOCKS - 1
    out = pl.pallas_call(
        body,
        grid=(N_BLOCKS + 1,),
        in_specs=[
            pl.BlockSpec((2 * S_HALF, K_SHARD), lambda k: (0, 0)),
            pl.BlockSpec((K_SHARD, BLK_N), lambda k: (0, jnp.minimum(k, last))),
        ],
        out_specs=pl.BlockSpec(
            (S_HALF, BLK_N), lambda k: (0, jnp.maximum(k - 1, 0))
        ),
        out_shape=jax.ShapeDtypeStruct((S_HALF, 8192), jnp.float32),
        scratch_shapes=[
            pltpu.VMEM((NSLOT, S_HALF, BLK_N), jnp.float32),
            pltpu.VMEM((NSLOT, S_HALF, BLK_N), jnp.float32),
            pltpu.VMEM((NSLOT, S_HALF, BLK_N), jnp.float32),
            pltpu.SemaphoreType.DMA((NSLOT,)),
            pltpu.SemaphoreType.DMA((NSLOT,)),
            pltpu.SemaphoreType.REGULAR,
        ],
        compiler_params=pltpu.CompilerParams(
            collective_id=0,
            dimension_semantics=("arbitrary",),
            vmem_limit_bytes=60 * 1024 * 1024,
        ),
    )(O2, Wo)
    return out.reshape(1, S_HALF, 8192)


# device time: 425550 ns/iter; 1.5510x vs baseline; 1.0671x over previous
import jax
import jax.numpy as jnp
from jax import lax
from jax.experimental import pallas as pl
from jax.experimental.pallas import tpu as pltpu

N_BLOCKS = 32
BLK_N = 8192 // N_BLOCKS
S_HALF = 1024
K_SHARD = 4096


def kernel(O, Wo):
    O2 = O.reshape(2048, K_SHARD).astype(jnp.bfloat16)
    NSLOT = 4

    def body(o_ref, wo_ref, out_ref,
             keep_buf, send_buf, recv_buf, send_sems, recv_sems, credit_sem):
        k = pl.program_id(0)
        my_x = lax.axis_index("x")
        my_y = lax.axis_index("y")
        my_z = lax.axis_index("z")
        partner = (my_x, 1 - my_y, my_z)

        slot = lax.rem(k, NSLOT)
        pslot = lax.rem(k + NSLOT - 1, NSLOT)

        def mk(s):
            return pltpu.make_async_remote_copy(
                src_ref=send_buf.at[s],
                dst_ref=recv_buf.at[s],
                send_sem=send_sems.at[s],
                recv_sem=recv_sems.at[s],
                device_id=partner,
                device_id_type=pl.DeviceIdType.MESH,
            )

        @pl.when(k == 0)
        def _():
            barrier_sem = pltpu.get_barrier_semaphore()
            pl.semaphore_signal(
                barrier_sem, inc=1,
                device_id=partner, device_id_type=pl.DeviceIdType.MESH,
            )
            pl.semaphore_wait(barrier_sem, 1)

        @pl.when(k < N_BLOCKS)
        def _():
            @pl.when(k >= NSLOT)
            def _():
                mk(slot).wait_send()

            keep_buf[slot] = jnp.dot(
                o_ref[pl.ds(S_HALF * my_y, S_HALF), :], wo_ref[...],
                preferred_element_type=jnp.float32,
            )
            send_buf[slot] = jnp.dot(
                o_ref[pl.ds(S_HALF * (1 - my_y), S_HALF), :], wo_ref[...],
                preferred_element_type=jnp.float32,
            )

            @pl.when(k >= NSLOT)
            def _():
                pl.semaphore_wait(credit_sem, 1)

            mk(slot).start()

        @pl.when(k >= 1)
        def _():
            mk(pslot).wait_recv()
            out_ref[...] = keep_buf[pslot] + recv_buf[pslot]
            pl.semaphore_signal(
                credit_sem, inc=1,
                device_id=partner, device_id_type=pl.DeviceIdType.MESH,
            )

        @pl.when(k == N_BLOCKS)
        def _():
            pl.semaphore_wait(credit_sem, NSLOT)
            for s in range(NSLOT):
                mk(s).wait_send()

    last = N_BLOCKS - 1
    out = pl.pallas_call(
        body,
        grid=(N_BLOCKS + 1,),
        in_specs=[
            pl.BlockSpec((2 * S_HALF, K_SHARD), lambda k: (0, 0)),
            pl.BlockSpec((K_SHARD, BLK_N), lambda k: (0, jnp.minimum(k, last))),
        ],
        out_specs=pl.BlockSpec(
            (S_HALF, BLK_N), lambda k: (0, jnp.maximum(k - 1, 0))
        ),
        out_shape=jax.ShapeDtypeStruct((S_HALF, 8192), jnp.float32),
        scratch_shapes=[
            pltpu.VMEM((NSLOT, S_HALF, BLK_N), jnp.float32),
            pltpu.VMEM((NSLOT, S_HALF, BLK_N), jnp.float32),
            pltpu.VMEM((NSLOT, S_HALF, BLK_N), jnp.float32),
            pltpu.SemaphoreType.DMA((NSLOT,)),
            pltpu.SemaphoreType.DMA((NSLOT,)),
            pltpu.SemaphoreType.REGULAR,
        ],
        compiler_params=pltpu.CompilerParams(
            collective_id=0,
            dimension_semantics=("arbitrary",),
            vmem_limit_bytes=60 * 1024 * 1024,
        ),
    )(O2, Wo)
    return out.reshape(1, S_HALF, 8192)


# device time: 260813 ns/iter; 2.5307x vs baseline; 1.6316x over previous
import jax
import jax.numpy as jnp
from jax import lax
from jax.experimental import pallas as pl
from jax.experimental.pallas import tpu as pltpu

N_BLOCKS = 16
BLK_N = 8192 // N_BLOCKS
QROWS = 256
K_SHARD = 4096
NSLOT = 4
MESH = pl.DeviceIdType.MESH


def kernel(O, Wo):
    O2 = O.reshape(2048, K_SHARD).astype(jnp.bfloat16)

    def body(o_ref, wo_ref, out_ref,
             kq, sq, yrecv, qbuf, qx, zrecv_a, zrecv_b,
             ysend_sems, yrecv_sems, xsend_sems, xrecv_sems,
             zasend_sems, zarecv_sems, zbsend_sems, zbrecv_sems,
             credit_y, credit_x, credit_z):
        s = pl.program_id(0)
        x = lax.axis_index("x")
        y = lax.axis_index("y")
        z = lax.axis_index("z")
        c = 2 * x + z
        y_partner = (x, 1 - y, z)
        x_partner = (1 - x, y, z)
        z_partner = (x, y, 1 - z)

        def rdma_y(sl):
            return pltpu.make_async_remote_copy(
                src_ref=sq.at[sl], dst_ref=yrecv.at[sl],
                send_sem=ysend_sems.at[sl], recv_sem=yrecv_sems.at[sl],
                device_id=y_partner, device_id_type=MESH)

        def rdma_x(sl):
            return pltpu.make_async_remote_copy(
                src_ref=qbuf.at[sl], dst_ref=qx.at[sl],
                send_sem=xsend_sems.at[sl], recv_sem=xrecv_sems.at[sl],
                device_id=x_partner, device_id_type=MESH)

        def rdma_za(sl):
            return pltpu.make_async_remote_copy(
                src_ref=qbuf.at[sl], dst_ref=zrecv_a.at[sl],
                send_sem=zasend_sems.at[sl], recv_sem=zarecv_sems.at[sl],
                device_id=z_partner, device_id_type=MESH)

        def rdma_zb(sl):
            return pltpu.make_async_remote_copy(
                src_ref=qx.at[sl], dst_ref=zrecv_b.at[sl],
                send_sem=zbsend_sems.at[sl], recv_sem=zbrecv_sems.at[sl],
                device_id=z_partner, device_id_type=MESH)

        @pl.when(s == 0)
        def _():
            bar = pltpu.get_barrier_semaphore()
            for nbr in (y_partner, x_partner, z_partner):
                pl.semaphore_signal(bar, inc=1, device_id=nbr,
                                    device_id_type=MESH)
            pl.semaphore_wait(bar, 3)

        @pl.when(s < N_BLOCKS)
        def _():
            sl = lax.rem(s, NSLOT)

            @pl.when(s >= NSLOT)
            def _():
                rdma_y(sl).wait_send()

            kq[sl] = jnp.dot(
                o_ref[pl.ds(1024 * y + QROWS * c, QROWS), :], wo_ref[...],
                preferred_element_type=jnp.float32)
            sq[sl] = jnp.dot(
                o_ref[pl.ds(1024 * (1 - y) + QROWS * c, QROWS), :],
                wo_ref[...], preferred_element_type=jnp.float32)

            @pl.when(s >= NSLOT)
            def _():
                pl.semaphore_wait(credit_y, 1)

            rdma_y(sl).start()

        @pl.when(jnp.logical_and(s >= 1, s <= N_BLOCKS))
        def _():
            b = s - 1
            sl = lax.rem(b, NSLOT)
            rdma_y(sl).wait_recv()

            @pl.when(b >= NSLOT)
            def _():
                rdma_x(sl).wait_send()

            qbuf[sl] = kq[sl] + yrecv[sl]
            pl.semaphore_signal(credit_y, inc=1, device_id=y_partner,
                                device_id_type=MESH)

            @pl.when(b >= NSLOT)
            def _():
                pl.semaphore_wait(credit_x, 1)

            rdma_x(sl).start()

        @pl.when(jnp.logical_and(s >= 2, s <= N_BLOCKS + 1))
        def _():
            b = s - 2
            sl = lax.rem(b, NSLOT)
            rdma_x(sl).wait_recv()

            @pl.when(b >= NSLOT)
            def _():
                pl.semaphore_wait(credit_z, 1)

            rdma_za(sl).start()
            rdma_zb(sl).start()

        @pl.when(s >= 3)
        def _():
            b = s - 3
            sl = lax.rem(b, NSLOT)
            rdma_za(sl).wait_recv()
            rdma_zb(sl).wait_recv()
            rdma_za(sl).wait_send()
            rdma_zb(sl).wait_send()

            out_ref[pl.ds(QROWS * c, QROWS), :] = qbuf[sl]
            out_ref[pl.ds(QROWS * jnp.bitwise_xor(c, 2), QROWS), :] = qx[sl]
            out_ref[pl.ds(QROWS * jnp.bitwise_xor(c, 1), QROWS), :] = (
                zrecv_a[sl])
            out_ref[pl.ds(QROWS * jnp.bitwise_xor(c, 3), QROWS), :] = (
                zrecv_b[sl])

            pl.semaphore_signal(credit_x, inc=1, device_id=x_partner,
                                device_id_type=MESH)
            pl.semaphore_signal(credit_z, inc=1, device_id=z_partner,
                                device_id_type=MESH)

        @pl.when(s == N_BLOCKS + 2)
        def _():
            pl.semaphore_wait(credit_y, NSLOT)
            pl.semaphore_wait(credit_x, NSLOT)
            pl.semaphore_wait(credit_z, NSLOT)
            for i in range(NSLOT):
                rdma_y(i).wait_send()
                rdma_x(i).wait_send()

    last = N_BLOCKS - 1
    out = pl.pallas_call(
        body,
        grid=(N_BLOCKS + 3,),
        in_specs=[
            pl.BlockSpec((2048, K_SHARD), lambda s: (0, 0)),
            pl.BlockSpec((K_SHARD, BLK_N), lambda s: (0, jnp.minimum(s, last))),
        ],
        out_specs=pl.BlockSpec(
            (4 * QROWS, BLK_N), lambda s: (0, jnp.maximum(s - 3, 0))
        ),
        out_shape=jax.ShapeDtypeStruct((4 * QROWS, 8192), jnp.float32),
        scratch_shapes=[
            pltpu.VMEM((NSLOT, QROWS, BLK_N), jnp.float32),
            pltpu.VMEM((NSLOT, QROWS, BLK_N), jnp.float32),
            pltpu.VMEM((NSLOT, QROWS, BLK_N), jnp.float32),
            pltpu.VMEM((NSLOT, QROWS, BLK_N), jnp.float32),
            pltpu.VMEM((NSLOT, QROWS, BLK_N), jnp.float32),
            pltpu.VMEM((NSLOT, QROWS, BLK_N), jnp.float32),
            pltpu.VMEM((NSLOT, QROWS, BLK_N), jnp.float32),
            pltpu.SemaphoreType.DMA((NSLOT,)),
            pltpu.SemaphoreType.DMA((NSLOT,)),
            pltpu.SemaphoreType.DMA((NSLOT,)),
            pltpu.SemaphoreType.DMA((NSLOT,)),
            pltpu.SemaphoreType.DMA((NSLOT,)),
            pltpu.SemaphoreType.DMA((NSLOT,)),
            pltpu.SemaphoreType.DMA((NSLOT,)),
            pltpu.SemaphoreType.DMA((NSLOT,)),
            pltpu.SemaphoreType.REGULAR,
            pltpu.SemaphoreType.REGULAR,
            pltpu.SemaphoreType.REGULAR,
        ],
        compiler_params=pltpu.CompilerParams(
            collective_id=0,
            dimension_semantics=("arbitrary",),
            vmem_limit_bytes=60 * 1024 * 1024,
        ),
    )(O2, Wo)
    return out.reshape(1, 4 * QROWS, 8192)


# device time: 246069 ns/iter; 2.6824x vs baseline; 1.0599x over previous
import jax
import jax.numpy as jnp
from jax import lax
from jax.experimental import pallas as pl
from jax.experimental.pallas import tpu as pltpu

N_BLOCKS = 16
BLK_N = 8192 // N_BLOCKS
QROWS = 256
K_SHARD = 4096
NSLOT = 4
MESH = pl.DeviceIdType.MESH


def kernel(O, Wo):
    O2 = O.reshape(2048, K_SHARD).astype(jnp.bfloat16)

    def body(o_ref, wo_ref, out_ref,
             kq, sq, yrecv, qbuf, q1recv, q2recv_a, q2recv_b,
             ysend_sems, yrecv_sems, fsend_sems, frecv_sems,
             sasend_sems, sarecv_sems, sbsend_sems, sbrecv_sems,
             credit_y, credit_1e, credit_1o, credit_2e, credit_2o):
        s = pl.program_id(0)
        x = lax.axis_index("x")
        y = lax.axis_index("y")
        z = lax.axis_index("z")
        c = 2 * x + z
        y_partner = (x, 1 - y, z)

        def plane_partners(b):
            p = lax.rem(b, 2)
            first = (jnp.where(p == 0, 1 - x, x), y,
                     jnp.where(p == 0, z, 1 - z))
            second = (jnp.where(p == 0, x, 1 - x), y,
                      jnp.where(p == 0, 1 - z, z))
            return first, second

        def rdma_y(sl):
            return pltpu.make_async_remote_copy(
                src_ref=sq.at[sl], dst_ref=yrecv.at[sl],
                send_sem=ysend_sems.at[sl], recv_sem=yrecv_sems.at[sl],
                device_id=y_partner, device_id_type=MESH)

        def rdma_first(sl, b):
            first, _ = plane_partners(b)
            return pltpu.make_async_remote_copy(
                src_ref=qbuf.at[sl], dst_ref=q1recv.at[sl],
                send_sem=fsend_sems.at[sl], recv_sem=frecv_sems.at[sl],
                device_id=first, device_id_type=MESH)

        def rdma_2a(sl, b):
            _, second = plane_partners(b)
            return pltpu.make_async_remote_copy(
                src_ref=qbuf.at[sl], dst_ref=q2recv_a.at[sl],
                send_sem=sasend_sems.at[sl], recv_sem=sarecv_sems.at[sl],
                device_id=second, device_id_type=MESH)

        def rdma_2b(sl, b):
            _, second = plane_partners(b)
            return pltpu.make_async_remote_copy(
                src_ref=q1recv.at[sl], dst_ref=q2recv_b.at[sl],
                send_sem=sbsend_sems.at[sl], recv_sem=sbrecv_sems.at[sl],
                device_id=second, device_id_type=MESH)

        @pl.when(s == 0)
        def _():
            bar = pltpu.get_barrier_semaphore()
            for nbr in (y_partner, (1 - x, y, z), (x, y, 1 - z)):
                pl.semaphore_signal(bar, inc=1, device_id=nbr,
                                    device_id_type=MESH)
            pl.semaphore_wait(bar, 3)

        @pl.when(s < N_BLOCKS)
        def _():
            sl = lax.rem(s, NSLOT)

            @pl.when(s >= NSLOT)
            def _():
                rdma_y(sl).wait_send()

            kq[sl] = jnp.dot(
                o_ref[pl.ds(1024 * y + QROWS * c, QROWS), :], wo_ref[...],
                preferred_element_type=jnp.float32)
            sq[sl] = jnp.dot(
                o_ref[pl.ds(1024 * (1 - y) + QROWS * c, QROWS), :],
                wo_ref[...], preferred_element_type=jnp.float32)

            @pl.when(s >= NSLOT)
            def _():
                pl.semaphore_wait(credit_y, 1)

            rdma_y(sl).start()

        @pl.when(jnp.logical_and(s >= 1, s <= N_BLOCKS))
        def _():
            b = s - 1
            sl = lax.rem(b, NSLOT)
            rdma_y(sl).wait_recv()

            @pl.when(b >= NSLOT)
            def _():
                rdma_first(sl, b).wait_send()

            qbuf[sl] = kq[sl] + yrecv[sl]
            pl.semaphore_signal(credit_y, inc=1, device_id=y_partner,
                                device_id_type=MESH)

            @pl.when(jnp.logical_and(b >= NSLOT, lax.rem(b, 2) == 0))
            def _():
                pl.semaphore_wait(credit_1e, 1)

            @pl.when(jnp.logical_and(b >= NSLOT, lax.rem(b, 2) == 1))
            def _():
                pl.semaphore_wait(credit_1o, 1)

            rdma_first(sl, b).start()

        @pl.when(jnp.logical_and(s >= 2, s <= N_BLOCKS + 1))
        def _():
            b = s - 2
            sl = lax.rem(b, NSLOT)
            rdma_first(sl, b).wait_recv()

            @pl.when(jnp.logical_and(b >= NSLOT, lax.rem(b, 2) == 0))
            def _():
                pl.semaphore_wait(credit_2e, 1)

            @pl.when(jnp.logical_and(b >= NSLOT, lax.rem(b, 2) == 1))
            def _():
                pl.semaphore_wait(credit_2o, 1)

            rdma_2a(sl, b).start()
            rdma_2b(sl, b).start()

        @pl.when(s >= 3)
        def _():
            b = s - 3
            sl = lax.rem(b, NSLOT)
            p = lax.rem(b, 2)
            m1 = jnp.where(p == 0, 2, 1)
            rdma_2a(sl, b).wait_recv()
            rdma_2b(sl, b).wait_recv()
            rdma_2a(sl, b).wait_send()
            rdma_2b(sl, b).wait_send()

            out_ref[pl.ds(QROWS * c, QROWS), :] = qbuf[sl]
            out_ref[pl.ds(QROWS * jnp.bitwise_xor(c, m1), QROWS), :] = (
                q1recv[sl])
            out_ref[pl.ds(QROWS * jnp.bitwise_xor(c, 3 - m1), QROWS), :] = (
                q2recv_a[sl])
            out_ref[pl.ds(QROWS * jnp.bitwise_xor(c, 3), QROWS), :] = (
                q2recv_b[sl])

            first, second = plane_partners(b)

            @pl.when(p == 0)
            def _():
                pl.semaphore_signal(credit_1e, inc=1, device_id=first,
                                    device_id_type=MESH)
                pl.semaphore_signal(credit_2e, inc=1, device_id=second,
                                    device_id_type=MESH)

            @pl.when(p == 1)
            def _():
                pl.semaphore_signal(credit_1o, inc=1, device_id=first,
                                    device_id_type=MESH)
                pl.semaphore_signal(credit_2o, inc=1, device_id=second,
                                    device_id_type=MESH)

        @pl.when(s == N_BLOCKS + 2)
        def _():
            pl.semaphore_wait(credit_y, NSLOT)
            pl.semaphore_wait(credit_1e, 2)
            pl.semaphore_wait(credit_1o, 2)
            pl.semaphore_wait(credit_2e, 2)
            pl.semaphore_wait(credit_2o, 2)
            for i in range(NSLOT):
                rdma_y(i).wait_send()
                rdma_first(i, i).wait_send()

    last = N_BLOCKS - 1
    out = pl.pallas_call(
        body,
        grid=(N_BLOCKS + 3,),
        in_specs=[
            pl.BlockSpec((2048, K_SHARD), lambda s: (0, 0)),
            pl.BlockSpec((K_SHARD, BLK_N), lambda s: (0, jnp.minimum(s, last))),
        ],
        out_specs=pl.BlockSpec(
            (4 * QROWS, BLK_N), lambda s: (0, jnp.maximum(s - 3, 0))
        ),
        out_shape=jax.ShapeDtypeStruct((4 * QROWS, 8192), jnp.float32),
        scratch_shapes=[
            pltpu.VMEM((NSLOT, QROWS, BLK_N), jnp.float32),
            pltpu.VMEM((NSLOT, QROWS, BLK_N), jnp.float32),
            pltpu.VMEM((NSLOT, QROWS, BLK_N), jnp.float32),
            pltpu.VMEM((NSLOT, QROWS, BLK_N), jnp.float32),
            pltpu.VMEM((NSLOT, QROWS, BLK_N), jnp.float32),
            pltpu.VMEM((NSLOT, QROWS, BLK_N), jnp.float32),
            pltpu.VMEM((NSLOT, QROWS, BLK_N), jnp.float32),
            pltpu.SemaphoreType.DMA((NSLOT,)),
            pltpu.SemaphoreType.DMA((NSLOT,)),
            pltpu.SemaphoreType.DMA((NSLOT,)),
            pltpu.SemaphoreType.DMA((NSLOT,)),
            pltpu.SemaphoreType.DMA((NSLOT,)),
            pltpu.SemaphoreType.DMA((NSLOT,)),
            pltpu.SemaphoreType.DMA((NSLOT,)),
            pltpu.SemaphoreType.DMA((NSLOT,)),
            pltpu.SemaphoreType.REGULAR,
            pltpu.SemaphoreType.REGULAR,
            pltpu.SemaphoreType.REGULAR,
            pltpu.SemaphoreType.REGULAR,
            pltpu.SemaphoreType.REGULAR,
        ],
        compiler_params=pltpu.CompilerParams(
            collective_id=0,
            dimension_semantics=("arbitrary",),
            vmem_limit_bytes=60 * 1024 * 1024,
        ),
    )(O2, Wo)
    return out.reshape(1, 4 * QROWS, 8192)


# device time: 241208 ns/iter; 2.7364x vs baseline; 1.0202x over previous
import jax
import jax.numpy as jnp
from jax import lax
from jax.experimental import pallas as pl
from jax.experimental.pallas import tpu as pltpu

N_BLOCKS = 16
BLK_N = 8192 // N_BLOCKS
QROWS = 256
K_SHARD = 4096
NSLOT = 4
MESH = pl.DeviceIdType.MESH


def kernel(O, Wo):
    O2 = O.reshape(2048, K_SHARD).astype(jnp.bfloat16)
    xi = lax.axis_index("x")
    zi = lax.axis_index("z")
    cq = 2 * xi + zi
    Oq = jnp.concatenate([
        lax.dynamic_slice(O2, (QROWS * cq, 0), (QROWS, K_SHARD)),
        lax.dynamic_slice(O2, (1024 + QROWS * cq, 0), (QROWS, K_SHARD)),
    ], axis=0)

    def body(o_ref, wo_ref, out_ref,
             pbuf, yrecv, qbuf, q1recv, q2recv_a, q2recv_b,
             ysend_sems, yrecv_sems, fsend_sems, frecv_sems,
             sasend_sems, sarecv_sems, sbsend_sems, sbrecv_sems,
             credit_y, credit_1e, credit_1o, credit_2e, credit_2o):
        s = pl.program_id(0)
        x = lax.axis_index("x")
        y = lax.axis_index("y")
        z = lax.axis_index("z")
        c = 2 * x + z
        y_partner = (x, 1 - y, z)

        def plane_partners(b):
            p = lax.rem(b, 2)
            first = (jnp.where(p == 0, 1 - x, x), y,
                     jnp.where(p == 0, z, 1 - z))
            second = (jnp.where(p == 0, x, 1 - x), y,
                      jnp.where(p == 0, 1 - z, z))
            return first, second

        def rdma_y(sl):
            return pltpu.make_async_remote_copy(
                src_ref=pbuf.at[sl, pl.ds(QROWS * (1 - y), QROWS)],
                dst_ref=yrecv.at[sl],
                send_sem=ysend_sems.at[sl], recv_sem=yrecv_sems.at[sl],
                device_id=y_partner, device_id_type=MESH)

        def rdma_first(sl, b):
            first, _ = plane_partners(b)
            return pltpu.make_async_remote_copy(
                src_ref=qbuf.at[sl], dst_ref=q1recv.at[sl],
                send_sem=fsend_sems.at[sl], recv_sem=frecv_sems.at[sl],
                device_id=first, device_id_type=MESH)

        def rdma_2a(sl, b):
            _, second = plane_partners(b)
            return pltpu.make_async_remote_copy(
                src_ref=qbuf.at[sl], dst_ref=q2recv_a.at[sl],
                send_sem=sasend_sems.at[sl], recv_sem=sarecv_sems.at[sl],
                device_id=second, device_id_type=MESH)

        def rdma_2b(sl, b):
            _, second = plane_partners(b)
            return pltpu.make_async_remote_copy(
                src_ref=q1recv.at[sl], dst_ref=q2recv_b.at[sl],
                send_sem=sbsend_sems.at[sl], recv_sem=sbrecv_sems.at[sl],
                device_id=second, device_id_type=MESH)

        @pl.when(s == 0)
        def _():
            bar = pltpu.get_barrier_semaphore()
            for nbr in (y_partner, (1 - x, y, z), (x, y, 1 - z)):
                pl.semaphore_signal(bar, inc=1, device_id=nbr,
                                    device_id_type=MESH)
            pl.semaphore_wait(bar, 3)

        @pl.when(s < N_BLOCKS)
        def _():
            sl = lax.rem(s, NSLOT)

            @pl.when(s >= NSLOT)
            def _():
                rdma_y(sl).wait_send()

            pbuf[sl] = jnp.dot(
                o_ref[...], wo_ref[...],
                preferred_element_type=jnp.float32)

            @pl.when(s >= NSLOT)
            def _():
                pl.semaphore_wait(credit_y, 1)

            rdma_y(sl).start()

        @pl.when(jnp.logical_and(s >= 1, s <= N_BLOCKS))
        def _():
            b = s - 1
            sl = lax.rem(b, NSLOT)
            rdma_y(sl).wait_recv()

            @pl.when(b >= NSLOT)
            def _():
                rdma_first(sl, b).wait_send()

            qbuf[sl] = pbuf[sl, pl.ds(QROWS * y, QROWS), :] + yrecv[sl]
            pl.semaphore_signal(credit_y, inc=1, device_id=y_partner,
                                device_id_type=MESH)

            @pl.when(jnp.logical_and(b >= NSLOT, lax.rem(b, 2) == 0))
            def _():
                pl.semaphore_wait(credit_1e, 1)

            @pl.when(jnp.logical_and(b >= NSLOT, lax.rem(b, 2) == 1))
            def _():
                pl.semaphore_wait(credit_1o, 1)

            rdma_first(sl, b).start()

        @pl.when(jnp.logical_and(s >= 2, s <= N_BLOCKS + 1))
        def _():
            b = s - 2
            sl = lax.rem(b, NSLOT)
            rdma_first(sl, b).wait_recv()

            @pl.when(jnp.logical_and(b >= NSLOT, lax.rem(b, 2) == 0))
            def _():
                pl.semaphore_wait(credit_2e, 1)

            @pl.when(jnp.logical_and(b >= NSLOT, lax.rem(b, 2) == 1))
            def _():
                pl.semaphore_wait(credit_2o, 1)

            rdma_2a(sl, b).start()
            rdma_2b(sl, b).start()

        @pl.when(s >= 3)
        def _():
            b = s - 3
            sl = lax.rem(b, NSLOT)
            p = lax.rem(b, 2)
            m1 = jnp.where(p == 0, 2, 1)
            rdma_2a(sl, b).wait_recv()
            rdma_2b(sl, b).wait_recv()
            rdma_2a(sl, b).wait_send()
            rdma_2b(sl, b).wait_send()

            out_ref[pl.ds(QROWS * c, QROWS), :] = qbuf[sl]
            out_ref[pl.ds(QROWS * jnp.bitwise_xor(c, m1), QROWS), :] = (
                q1recv[sl])
            out_ref[pl.ds(QROWS * jnp.bitwise_xor(c, 3 - m1), QROWS), :] = (
                q2recv_a[sl])
            out_ref[pl.ds(QROWS * jnp.bitwise_xor(c, 3), QROWS), :] = (
                q2recv_b[sl])

            first, second = plane_partners(b)

            @pl.when(p == 0)
            def _():
                pl.semaphore_signal(credit_1e, inc=1, device_id=first,
                                    device_id_type=MESH)
                pl.semaphore_signal(credit_2e, inc=1, device_id=second,
                                    device_id_type=MESH)

            @pl.when(p == 1)
            def _():
                pl.semaphore_signal(credit_1o, inc=1, device_id=first,
                                    device_id_type=MESH)
                pl.semaphore_signal(credit_2o, inc=1, device_id=second,
                                    device_id_type=MESH)

        @pl.when(s == N_BLOCKS + 2)
        def _():
            pl.semaphore_wait(credit_y, NSLOT)
            pl.semaphore_wait(credit_1e, 2)
            pl.semaphore_wait(credit_1o, 2)
            pl.semaphore_wait(credit_2e, 2)
            pl.semaphore_wait(credit_2o, 2)
            for i in range(NSLOT):
                rdma_y(i).wait_send()
                rdma_first(i, i).wait_send()

    last = N_BLOCKS - 1
    out = pl.pallas_call(
        body,
        grid=(N_BLOCKS + 3,),
        in_specs=[
            pl.BlockSpec((2 * QROWS, K_SHARD), lambda s: (0, 0)),
            pl.BlockSpec((K_SHARD, BLK_N), lambda s: (0, jnp.minimum(s, last))),
        ],
        out_specs=pl.BlockSpec(
            (4 * QROWS, BLK_N), lambda s: (0, jnp.maximum(s - 3, 0))
        ),
        out_shape=jax.ShapeDtypeStruct((4 * QROWS, 8192), jnp.float32),
        scratch_shapes=[
            pltpu.VMEM((NSLOT, 2 * QROWS, BLK_N), jnp.float32),
            pltpu.VMEM((NSLOT, QROWS, BLK_N), jnp.float32),
            pltpu.VMEM((NSLOT, QROWS, BLK_N), jnp.float32),
            pltpu.VMEM((NSLOT, QROWS, BLK_N), jnp.float32),
            pltpu.VMEM((NSLOT, QROWS, BLK_N), jnp.float32),
            pltpu.VMEM((NSLOT, QROWS, BLK_N), jnp.float32),
            pltpu.SemaphoreType.DMA((NSLOT,)),
            pltpu.SemaphoreType.DMA((NSLOT,)),
            pltpu.SemaphoreType.DMA((NSLOT,)),
            pltpu.SemaphoreType.DMA((NSLOT,)),
            pltpu.SemaphoreType.DMA((NSLOT,)),
            pltpu.SemaphoreType.DMA((NSLOT,)),
            pltpu.SemaphoreType.DMA((NSLOT,)),
            pltpu.SemaphoreType.DMA((NSLOT,)),
            pltpu.SemaphoreType.REGULAR,
            pltpu.SemaphoreType.REGULAR,
            pltpu.SemaphoreType.REGULAR,
            pltpu.SemaphoreType.REGULAR,
            pltpu.SemaphoreType.REGULAR,
        ],
        compiler_params=pltpu.CompilerParams(
            collective_id=0,
            dimension_semantics=("arbitrary",),
            vmem_limit_bytes=60 * 1024 * 1024,
        ),
    )(Oq, Wo)
    return out.reshape(1, 4 * QROWS, 8192)
